# baseline (speedup 1.0000x reference)
import math

import jax
import jax.numpy as jnp
from jax import lax
from jax.experimental import pallas as pl
from jax.experimental.pallas import tpu as pltpu
from jax.experimental.pallas import tpu_sc as plsc

NC = 2
NS = 16
L = 16
NW = NC * NS

B = 4
T = 8192
D = 768
SCALE = math.sqrt(float(D))

TPW = T // NW
CH = 16
NTC = TPW // CH
VPR = D // L


def _emb_kernel(ids_hbm, tok_hbm, pos_hbm, out_hbm,
                idx_v, tok0, tok1, tok2, tok3, pos0, pos1,
                gs0, gs1, gs2, gs3, ps0, ps1, os0, os1, os2, os3):
    wid = lax.axis_index("s") * NC + lax.axis_index("c")
    t0 = wid * TPW

    for b in range(B):
        pltpu.sync_copy(ids_hbm.at[pl.ds(b * T + t0, TPW)],
                        idx_v.at[pl.ds(b * TPW, TPW)])

    toks = (tok0, tok1, tok2, tok3)
    poss = (pos0, pos1)
    gsems = (gs0, gs1, gs2, gs3)
    psems = (ps0, ps1)
    osems = (os0, os1, os2, os3)

    pltpu.async_copy(pos_hbm.at[pl.ds(t0, CH)], pos0, ps0)
    pltpu.async_copy(tok_hbm.at[idx_v.at[pl.ds(0, CH)]], tok0, gs0)
    pltpu.async_copy(tok_hbm.at[idx_v.at[pl.ds(TPW, CH)]], tok1, gs1)

    @pl.loop(0, NTC, step=2)
    def _tc2(tc0):
        for tcu in range(2):
            tc = tc0 + tcu
            posbuf, psem = poss[tcu], psems[tcu]
            nposbuf, npsem = poss[1 - tcu], psems[1 - tcu]
            for b in range(B):
                u = b
                w = (b + 2) % 4

                if b < 2:
                    @pl.when(tc > 0)
                    def _():
                        pltpu.make_async_copy(
                            toks[w], out_hbm.at[pl.ds(0, CH)],
                            osems[w]).wait()
                    pltpu.async_copy(
                        tok_hbm.at[idx_v.at[pl.ds((b + 2) * TPW + tc * CH,
                                                  CH)]],
                        toks[w], gsems[w])
                else:
                    @pl.when(tc < NTC - 1)
                    def _():
                        pltpu.make_async_copy(
                            toks[w], out_hbm.at[pl.ds(0, CH)],
                            osems[w]).wait()
                        pltpu.async_copy(
                            tok_hbm.at[idx_v.at[pl.ds(
                                (b - 2) * TPW + (tc + 1) * CH, CH)]],
                            toks[w], gsems[w])

                if b == 1:
                    @pl.when(tc < NTC - 1)
                    def _():
                        pltpu.async_copy(
                            pos_hbm.at[pl.ds(t0 + (tc + 1) * CH, CH)],
                            nposbuf, npsem)

                pltpu.make_async_copy(
                    tok_hbm.at[pl.ds(0, CH)], toks[u], gsems[u]).wait()
                if b == 0:
                    pltpu.make_async_copy(
                        pos_hbm.at[pl.ds(0, CH)], posbuf, psem).wait()

                tbuf = toks[u]

                @pl.loop(0, CH)
                def _row(r):
                    for k in range(VPR):
                        sl = pl.ds(k * L, L)
                        tbuf[r, sl] = tbuf[r, sl] * SCALE + posbuf[r, sl]

                pltpu.async_copy(
                    tbuf, out_hbm.at[pl.ds(b * T + t0 + tc * CH, CH)],
                    osems[u])

    for u in range(4):
        pltpu.make_async_copy(toks[u], out_hbm.at[pl.ds(0, CH)],
                              osems[u]).wait()


@jax.jit
def _emb_call(ids_flat, token_embed, pos_embed):
    mesh = plsc.VectorSubcoreMesh(core_axis_name="c", subcore_axis_name="s")
    fn = pl.kernel(
        _emb_kernel,
        out_type=jax.ShapeDtypeStruct((B * T, D), jnp.float32),
        mesh=mesh,
        scratch_types=[
            pltpu.VMEM((B * TPW,), jnp.int32),
            pltpu.VMEM((CH, D), jnp.float32),
            pltpu.VMEM((CH, D), jnp.float32),
            pltpu.VMEM((CH, D), jnp.float32),
            pltpu.VMEM((CH, D), jnp.float32),
            pltpu.VMEM((CH, D), jnp.float32),
            pltpu.VMEM((CH, D), jnp.float32),
            pltpu.SemaphoreType.DMA,
            pltpu.SemaphoreType.DMA,
            pltpu.SemaphoreType.DMA,
            pltpu.SemaphoreType.DMA,
            pltpu.SemaphoreType.DMA,
            pltpu.SemaphoreType.DMA,
            pltpu.SemaphoreType.DMA,
            pltpu.SemaphoreType.DMA,
            pltpu.SemaphoreType.DMA,
            pltpu.SemaphoreType.DMA,
        ],
    )
    return fn(ids_flat, token_embed, pos_embed)


def kernel(input_ids, token_embed, pos_embed):
    ids_flat = input_ids.astype(jnp.int32).reshape(B * T)
    out = _emb_call(ids_flat, token_embed, pos_embed)
    return out.reshape(B, T, D)
